# Initial kernel scaffold; baseline (speedup 1.0000x reference)
import jax
import jax.numpy as jnp
from jax.experimental import pallas as pl


def kernel(edge_index, features, W, b):
    raise NotImplementedError("write your pallas kernel here")


# speedup vs baseline: 3.1137x; 3.1137x over previous
import functools

import jax
import jax.numpy as jnp
from jax import lax
from jax.experimental import pallas as pl
from jax.experimental.pallas import tpu as pltpu
from jax.experimental.pallas import tpu_sc as plsc

NC = 2
NS = 16
CH = 128
NQ = 4


def _sc_aggregate(n_acc, dq, ts):
  rt = n_acc // NS

  mesh = plsc.VectorSubcoreMesh(core_axis_name="c", subcore_axis_name="s")

  @functools.partial(
      pl.kernel,
      out_type=[jax.ShapeDtypeStruct((n_acc, dq), jnp.float32)
                for _ in range(NQ)],
      mesh=mesh,
      scratch_types=[
          pltpu.VMEM((ts, CH), jnp.int32),
          pltpu.VMEM((ts, CH), jnp.int32),
          pltpu.VMEM((2, CH, dq), jnp.float32),
          pltpu.VMEM_SHARED((n_acc, dq), jnp.float32),
          pltpu.SemaphoreType.DMA,
          pltpu.SemaphoreType.DMA,
      ],
      compiler_params=pltpu.CompilerParams(use_tc_tiling_on_sc=False),
  )
  def agg(colh, rowh, f0, f1, f2, f3, zrows,
          o0, o1, o2, o3, colv, rowv, gbuf, acc, sem0, sem1):
    cid = lax.axis_index("c")
    sid = lax.axis_index("s")
    sems = (sem0, sem1)
    srcs = (f0, f1, f2, f3)
    outs = (o0, o1, o2, o3)

    pltpu.sync_copy(colh.at[sid], colv)
    pltpu.sync_copy(rowh.at[sid], rowv)

    for p in range(2):
      zr = rt // 8
      for z in range(8):
        pltpu.sync_copy(zrows, acc.at[pl.ds(sid * rt + z * zr, zr)])
      plsc.subcore_barrier()

      def body(j, carry):
        @pl.when(cid == 0)
        def _():
          pltpu.async_copy(srcs[p].at[colv.at[j]], gbuf.at[0], sem0).wait()

        @pl.when(cid == 1)
        def _():
          pltpu.async_copy(srcs[2 + p].at[colv.at[j]], gbuf.at[0],
                           sem0).wait()

        pltpu.sync_copy(gbuf.at[0], acc.at[rowv.at[j]], add=True)
        return carry

      lax.fori_loop(0, ts, body, 0)

      plsc.subcore_barrier()

      @pl.when(cid == 0)
      def _():
        pltpu.sync_copy(acc.at[pl.ds(sid * rt, rt)],
                        outs[p].at[pl.ds(sid * rt, rt)])

      @pl.when(cid == 1)
      def _():
        pltpu.sync_copy(acc.at[pl.ds(sid * rt, rt)],
                        outs[2 + p].at[pl.ds(sid * rt, rt)])

  return agg


def _tc_linear_body(a0, a1, a2, a3, w0, w1, w2, w3, bb, out):
  acc = jnp.dot(a0[...], w0[...], preferred_element_type=jnp.float32)
  acc += jnp.dot(a1[...], w1[...], preferred_element_type=jnp.float32)
  acc += jnp.dot(a2[...], w2[...], preferred_element_type=jnp.float32)
  acc += jnp.dot(a3[...], w3[...], preferred_element_type=jnp.float32)
  out[...] = acc + bb[...]


def kernel(edge_index, features, W, b):
  n, d = features.shape
  d_out = W.shape[0]
  e = edge_index.shape[1]
  dq = d // NQ

  ts = -(-e // (NS * CH))
  e_pad = NS * ts * CH
  n_acc = -(-n // (NS * 8)) * (NS * 8)

  row = edge_index[0].astype(jnp.int32)
  col = edge_index[1].astype(jnp.int32)
  row = jnp.pad(row, (0, e_pad - e), constant_values=n)
  col = jnp.pad(col, (0, e_pad - e), constant_values=0)
  row3 = row.reshape(NS, ts, CH)
  col3 = col.reshape(NS, ts, CH)

  fq = [features[:, q * dq:(q + 1) * dq] for q in range(NQ)]
  zrows = jnp.zeros((n_acc // NS // 8, dq), jnp.float32)

  aggs = _sc_aggregate(n_acc, dq, ts)(col3, row3, *fq, zrows)

  blk = 1000
  grid = n // blk
  wq = [W[:, q * dq:(q + 1) * dq].T for q in range(NQ)]
  bb = b.reshape(1, d_out)

  out = pl.pallas_call(
      _tc_linear_body,
      grid=(grid,),
      in_specs=(
          [pl.BlockSpec((blk, dq), lambda i: (i, 0)) for _ in range(NQ)]
          + [pl.BlockSpec((dq, d_out), lambda i: (0, 0)) for _ in range(NQ)]
          + [pl.BlockSpec((1, d_out), lambda i: (0, 0))]
      ),
      out_specs=pl.BlockSpec((blk, d_out), lambda i: (i, 0)),
      out_shape=jax.ShapeDtypeStruct((n, d_out), jnp.float32),
  )(*aggs, *wq, bb)

  return out
